# baseline (speedup 1.0000x reference)
import dataclasses
import functools

import jax
import jax.numpy as jnp
from jax import lax
from jax.experimental import pallas as pl
from jax.experimental.pallas import tpu as pltpu
from jax.experimental.pallas import tpu_sc as plsc

_N = 10000
_E = 320000
_D = 128
_DE = 16
_H = 128

_NC = 2
_NS = 16
_NL = 16
_NW = _NC * _NS
_CH = 64
_TPT = 158
_EPAD = _CH * _NW * _TPT
_NPAD = 10112
_RPT = _NPAD // _NS


def _dense_in(x, W, al, ar):
    def body(x_ref, w_ref, a_ref, ft_ref, el_ref, er_ref):
        ft = jnp.dot(x_ref[...], w_ref[...], preferred_element_type=jnp.float32)
        ft_ref[...] = ft
        el_ref[...] = jnp.dot(ft, a_ref[...][:, 0:1])
        er_ref[...] = jnp.dot(ft, a_ref[...][:, 1:2])

    a = jnp.stack([al, ar], axis=1)
    ft, el, er = pl.pallas_call(
        body,
        out_shape=(
            jax.ShapeDtypeStruct((_N, _H), jnp.float32),
            jax.ShapeDtypeStruct((_N, 1), jnp.float32),
            jax.ShapeDtypeStruct((_N, 1), jnp.float32),
        ),
    )(x, W, a)
    return ft, el[:, 0], er[:, 0]


def _eterm(edge_in, We, ae):
    def body(e_ref, w_ref, ae_ref, out_ref):
        w = jnp.dot(w_ref[...], ae_ref[...])
        wfull = jnp.concatenate([w] * 8, axis=0)
        ic = jax.lax.broadcasted_iota(jnp.int32, (8 * _DE, 8), 0)
        ik = jax.lax.broadcasted_iota(jnp.int32, (8 * _DE, 8), 1)
        wtile = jnp.where((ic // _DE) == ik, wfull, 0.0)
        out_ref[...] = jnp.dot(e_ref[...], wtile,
                               preferred_element_type=jnp.float32)

    out = pl.pallas_call(
        body,
        out_shape=jax.ShapeDtypeStruct((_E // 8, 8), jnp.float32),
    )(edge_in.reshape(_E // 8, 8 * _DE), We, ae.reshape(_H, 1))
    return out.reshape(_E)


def _combine(acc128, acc32, We, b, apply_tanh):
    def body(a128_ref, a32_ref, we_ref, b_ref, h_ref):
        num = a128_ref[0, :_N] + a128_ref[1, :_N]
        s32 = a32_ref[0, :_N] + a32_ref[1, :_N]
        se = s32[:, 0:_DE]
        den = s32[:, _DE:_DE + 1]
        h = (num + jnp.dot(se, we_ref[...], preferred_element_type=jnp.float32))
        h = h / jnp.maximum(den, 1e-9) + b_ref[...]
        if apply_tanh:
            h = jnp.tanh(h)
        h_ref[...] = h

    return pl.pallas_call(
        body,
        out_shape=jax.ShapeDtypeStruct((_N, _H), jnp.float32),
    )(acc128, acc32, We, b.reshape(1, _H))


def _edge_pass(ft, el, er, eterm, ein2, src, dst):
    mesh = plsc.VectorSubcoreMesh(core_axis_name="c", subcore_axis_name="s")
    cp = pltpu.CompilerParams(use_tc_tiling_on_sc=False)
    if "needs_layout_passes" in pltpu.CompilerParams.__dataclass_fields__:
        cp = dataclasses.replace(cp, needs_layout_passes=False)

    slot_scratch = [
        pltpu.VMEM((_CH,), jnp.int32),
        pltpu.VMEM((_CH,), jnp.int32),
        pltpu.VMEM((_CH,), jnp.int32),
        pltpu.VMEM((_CH,), jnp.float32),
        pltpu.VMEM((_CH // 8, _H), jnp.float32),
        pltpu.VMEM((_CH,), jnp.float32),
        pltpu.VMEM((_CH,), jnp.float32),
        pltpu.VMEM((_CH,), jnp.float32),
        pltpu.VMEM((_CH, _H), jnp.float32),
        pltpu.VMEM((_CH, 32), jnp.float32),
        pltpu.SemaphoreType.DMA,
        pltpu.SemaphoreType.DMA,
        pltpu.SemaphoreType.DMA,
    ]

    @functools.partial(
        pl.kernel,
        out_type=(jax.ShapeDtypeStruct((_NC, _NPAD, _H), jnp.float32),
                  jax.ShapeDtypeStruct((_NC, _NPAD, 32), jnp.float32)),
        mesh=mesh,
        compiler_params=cp,
        scratch_types=[
            pltpu.VMEM_SHARED((_NPAD, _H), jnp.float32),
            pltpu.VMEM_SHARED((_NPAD, 32), jnp.float32),
        ] + slot_scratch + slot_scratch,
    )
    def k(ft_hbm, el_hbm, er_hbm, et_hbm, ein_hbm, src_hbm, dst_hbm,
          out128_hbm, out32_hbm, acc128, acc32, *slotrefs):
        c = lax.axis_index("c")
        s = lax.axis_index("s")
        wid = s * _NC + c
        zv = jnp.zeros((_NL,), jnp.float32)
        nslot = len(slot_scratch)
        slots = [slotrefs[:nslot], slotrefs[nslot:]]

        def issue_a(t, sl):
            src_v, dst_v, _, et_v, ein_v = sl[0], sl[1], sl[2], sl[3], sl[4]
            sem_a = sl[10]
            g = (wid + t * _NW) * _CH
            pltpu.async_copy(src_hbm.at[pl.ds(g, _CH)], src_v, sem_a)
            pltpu.async_copy(dst_hbm.at[pl.ds(g, _CH)], dst_v, sem_a)
            pltpu.async_copy(et_hbm.at[pl.ds(g, _CH)], et_v, sem_a)
            pltpu.async_copy(
                ein_hbm.at[pl.ds((wid + t * _NW) * (_CH // 8), _CH // 8)],
                ein_v, sem_a)

        def wait_a(sl):
            src_v, dst_v, _, et_v, ein_v = sl[0], sl[1], sl[2], sl[3], sl[4]
            sem_a = sl[10]
            pltpu.make_async_copy(src_hbm.at[pl.ds(0, _CH)], src_v, sem_a).wait()
            pltpu.make_async_copy(dst_hbm.at[pl.ds(0, _CH)], dst_v, sem_a).wait()
            pltpu.make_async_copy(et_hbm.at[pl.ds(0, _CH)], et_v, sem_a).wait()
            pltpu.make_async_copy(
                ein_hbm.at[pl.ds(0, _CH // 8)], ein_v, sl[10]).wait()

        def issue_b(sl):
            src_v, dst_v, elg_v, erg_v, ftb = sl[0], sl[1], sl[5], sl[6], sl[8]
            sem_b = sl[11]
            pltpu.async_copy(el_hbm.at[src_v], elg_v, sem_b)
            pltpu.async_copy(er_hbm.at[dst_v], erg_v, sem_b)
            pltpu.async_copy(ft_hbm.at[src_v], ftb, sem_b)

        def wait_b(sl):
            src_v, dst_v, elg_v, erg_v, ftb = sl[0], sl[1], sl[5], sl[6], sl[8]
            sem_b = sl[11]
            pltpu.make_async_copy(el_hbm.at[src_v], elg_v, sem_b).wait()
            pltpu.make_async_copy(er_hbm.at[dst_v], erg_v, sem_b).wait()
            pltpu.make_async_copy(ft_hbm.at[src_v], ftb, sem_b).wait()

        iota = lax.iota(jnp.int32, _NL)
        one0 = jnp.where(iota == 0, jnp.float32(1), jnp.float32(0))

        def compute(sl):
            dst_v, dsts_v, et_v, ein_v = sl[1], sl[2], sl[3], sl[4]
            elg_v, erg_v, exb, ftb, st2 = sl[5], sl[6], sl[7], sl[8], sl[9]

            @pl.loop(0, _CH, step=_NL)
            def _ex16(i):
                e = elg_v[pl.ds(i, _NL)] + erg_v[pl.ds(i, _NL)]
                e = e + et_v[pl.ds(i, _NL)]
                e = jnp.maximum(e, 0.2 * e)
                exb[pl.ds(i, _NL)] = jnp.exp(e)
                dsts_v[pl.ds(i, _NL)] = dst_v[pl.ds(i, _NL)]

            @pl.loop(0, _CH)
            def _edge(i):
                bex = plsc.load_gather(exb, [jnp.broadcast_to(i, (_NL,))])
                for kk in range(_H // _NL):
                    slc = pl.ds(kk * _NL, _NL)
                    ftb[i, slc] = ftb[i, slc] * bex
                st2[i, pl.ds(0, _NL)] = (
                    ein_v[i // 8, pl.ds((i % 8) * _DE, _DE)] * bex)
                st2[i, pl.ds(_NL, _NL)] = bex * one0

        def issue_d(sl):
            dsts_v, ftb, st2, sem_d = sl[2], sl[8], sl[9], sl[12]
            pltpu.async_copy(ftb, acc128.at[dsts_v], sem_d, add=True)
            pltpu.async_copy(st2, acc32.at[dsts_v], sem_d, add=True)

        def wait_d(sl):
            dsts_v, ftb, st2, sem_d = sl[2], sl[8], sl[9], sl[12]
            pltpu.make_async_copy(ftb, acc128.at[dsts_v], sem_d).wait()
            pltpu.make_async_copy(st2, acc32.at[dsts_v], sem_d).wait()

        ftb0, st20 = slots[0][8], slots[0][9]

        @pl.loop(0, _CH)
        def _zrow(i):
            for kk in range(_H // _NL):
                ftb0[i, pl.ds(kk * _NL, _NL)] = zv
            st20[i, pl.ds(0, _NL)] = zv
            st20[i, pl.ds(_NL, _NL)] = zv

        r0 = s * _RPT
        for j in range(0, _RPT, _CH):
            nr = min(_CH, _RPT - j)
            pltpu.sync_copy(ftb0.at[pl.ds(0, nr)], acc128.at[pl.ds(r0 + j, nr)])
            pltpu.sync_copy(st20.at[pl.ds(0, nr)], acc32.at[pl.ds(r0 + j, nr)])
        plsc.subcore_barrier()

        issue_a(0, slots[0])
        issue_a(1, slots[1])
        wait_a(slots[0])
        issue_b(slots[0])

        @pl.loop(0, _TPT, step=2)
        def _pair(tt):
            for j in (0, 1):
                t = tt + j
                p, p1 = slots[j], slots[1 - j]
                if j == 0:
                    @pl.when(tt > 0)
                    def _():
                        wait_d(p1)
                else:
                    wait_d(p1)
                wait_a(p1)
                issue_b(p1)
                wait_b(p)
                compute(p)
                issue_d(p)
                issue_a(jnp.minimum(t + 2, _TPT - 1), p)

        wait_a(slots[1])
        wait_b(slots[0])
        wait_d(slots[1])

        plsc.subcore_barrier()
        pltpu.sync_copy(acc128.at[pl.ds(r0, _RPT)],
                        out128_hbm.at[c, pl.ds(r0, _RPT)])
        pltpu.sync_copy(acc32.at[pl.ds(r0, _RPT)],
                        out32_hbm.at[c, pl.ds(r0, _RPT)])

    return k(ft, el, er, eterm, ein2, src, dst)


def _layer(x, ein2, src, dst, W, We, al, ar, eterm, b, apply_tanh):
    ft, el, er = _dense_in(x, W, al, ar)
    acc128, acc32 = _edge_pass(ft, el, er, eterm, ein2, src, dst)
    return _combine(acc128, acc32, We, b, apply_tanh)


def kernel(node_in, edge_index, edge_in,
           W1, We1, al1, ar1, ae1, b1,
           W2, We2, al2, ar2, ae2, b2,
           W3, We3, al3, ar3, ae3, b3):
    npad = _EPAD - _E
    src = jnp.concatenate(
        [edge_index[0].astype(jnp.int32), jnp.zeros((npad,), jnp.int32)])
    dst = jnp.concatenate(
        [edge_index[1].astype(jnp.int32), jnp.zeros((npad,), jnp.int32)])
    ein2 = jnp.concatenate(
        [edge_in.reshape(_E // 8, 8 * _DE),
         jnp.zeros((npad // 8, 8 * _DE), jnp.float32)])
    etpad = jnp.full((npad,), -1e5, jnp.float32)
    et1 = jnp.concatenate([_eterm(edge_in, We1, ae1), etpad])
    et2 = jnp.concatenate([_eterm(edge_in, We2, ae2), etpad])
    et3 = jnp.concatenate([_eterm(edge_in, We3, ae3), etpad])
    h = _layer(node_in, ein2, src, dst, W1, We1, al1, ar1, et1, b1, True)
    h = _layer(h, ein2, src, dst, W2, We2, al2, ar2, et2, b2, True)
    h = _layer(h, ein2, src, dst, W2, We2, al2, ar2, et2, b2, True)
    h = _layer(h, ein2, src, dst, W3, We3, al3, ar3, et3, b3, False)
    return h


# speedup vs baseline: 1.1534x; 1.1534x over previous
import dataclasses
import functools

import jax
import jax.numpy as jnp
from jax import lax
from jax.experimental import pallas as pl
from jax.experimental.pallas import tpu as pltpu
from jax.experimental.pallas import tpu_sc as plsc

_N = 10000
_E = 320000
_D = 128
_DE = 16
_H = 128

_NC = 2
_NS = 16
_NL = 16
_NW = _NC * _NS
_CH = 64
_TPT = 158
_EPAD = _CH * _NW * _TPT
_NPAD = 10112
_RPT = _NPAD // _NS


def _dense_in(x, W, al, ar):
    def body(x_ref, w_ref, a_ref, ft_ref, el_ref, er_ref):
        ft = jnp.dot(x_ref[...], w_ref[...], preferred_element_type=jnp.float32)
        ft_ref[...] = ft
        el_ref[...] = jnp.dot(ft, a_ref[...][:, 0:1])
        er_ref[...] = jnp.dot(ft, a_ref[...][:, 1:2])

    a = jnp.stack([al, ar], axis=1)
    ft, el, er = pl.pallas_call(
        body,
        out_shape=(
            jax.ShapeDtypeStruct((_N, _H), jnp.float32),
            jax.ShapeDtypeStruct((_N, 1), jnp.float32),
            jax.ShapeDtypeStruct((_N, 1), jnp.float32),
        ),
    )(x, W, a)
    return ft, el[:, 0], er[:, 0]


def _eterm(edge_in, We, ae):
    def body(e_ref, w_ref, ae_ref, out_ref):
        w = jnp.dot(w_ref[...], ae_ref[...])
        wfull = jnp.concatenate([w] * 8, axis=0)
        ic = jax.lax.broadcasted_iota(jnp.int32, (8 * _DE, 8), 0)
        ik = jax.lax.broadcasted_iota(jnp.int32, (8 * _DE, 8), 1)
        wtile = jnp.where((ic // _DE) == ik, wfull, 0.0)
        out_ref[...] = jnp.dot(e_ref[...], wtile,
                               preferred_element_type=jnp.float32)

    out = pl.pallas_call(
        body,
        out_shape=jax.ShapeDtypeStruct((_E // 8, 8), jnp.float32),
    )(edge_in.reshape(_E // 8, 8 * _DE), We, ae.reshape(_H, 1))
    return out.reshape(_E)


def _combine(acc128, acc32, We, b, apply_tanh):
    def body(a128_ref, a32_ref, we_ref, b_ref, h_ref):
        num = a128_ref[0, :_N] + a128_ref[1, :_N]
        s32 = a32_ref[0, :_N] + a32_ref[1, :_N]
        se = s32[:, 0:_DE]
        den = s32[:, _DE:_DE + 1]
        h = (num + jnp.dot(se, we_ref[...], preferred_element_type=jnp.float32))
        h = h / jnp.maximum(den, 1e-9) + b_ref[...]
        if apply_tanh:
            h = jnp.tanh(h)
        h_ref[...] = h

    return pl.pallas_call(
        body,
        out_shape=jax.ShapeDtypeStruct((_N, _H), jnp.float32),
    )(acc128, acc32, We, b.reshape(1, _H))


def _edge_pass(ft, el, er, eterm, ein2, src, dst):
    mesh = plsc.VectorSubcoreMesh(core_axis_name="c", subcore_axis_name="s")
    cp = pltpu.CompilerParams(use_tc_tiling_on_sc=False)
    if "needs_layout_passes" in pltpu.CompilerParams.__dataclass_fields__:
        cp = dataclasses.replace(cp, needs_layout_passes=False)

    slot_scratch = [
        pltpu.VMEM((_CH,), jnp.int32),
        pltpu.VMEM((_CH,), jnp.int32),
        pltpu.VMEM((_CH,), jnp.int32),
        pltpu.VMEM((_CH,), jnp.float32),
        pltpu.VMEM((_CH // 8, _H), jnp.float32),
        pltpu.VMEM((_CH,), jnp.float32),
        pltpu.VMEM((_CH,), jnp.float32),
        pltpu.VMEM((_CH,), jnp.float32),
        pltpu.VMEM((_CH, _H), jnp.float32),
        pltpu.VMEM((_CH, 32), jnp.float32),
        pltpu.SemaphoreType.DMA,
        pltpu.SemaphoreType.DMA,
        pltpu.SemaphoreType.DMA,
    ]

    @functools.partial(
        pl.kernel,
        out_type=(jax.ShapeDtypeStruct((_NC, _NPAD, _H), jnp.float32),
                  jax.ShapeDtypeStruct((_NC, _NPAD, 32), jnp.float32)),
        mesh=mesh,
        compiler_params=cp,
        scratch_types=[
            pltpu.VMEM_SHARED((_NPAD, _H), jnp.float32),
            pltpu.VMEM_SHARED((_NPAD, 32), jnp.float32),
        ] + slot_scratch + slot_scratch,
    )
    def k(ft_hbm, el_hbm, er_hbm, et_hbm, ein_hbm, src_hbm, dst_hbm,
          out128_hbm, out32_hbm, acc128, acc32, *slotrefs):
        c = lax.axis_index("c")
        s = lax.axis_index("s")
        wid = s * _NC + c
        zv = jnp.zeros((_NL,), jnp.float32)
        nslot = len(slot_scratch)
        slots = [slotrefs[:nslot], slotrefs[nslot:]]

        def issue_a(t, sl):
            src_v, dst_v, _, et_v, ein_v = sl[0], sl[1], sl[2], sl[3], sl[4]
            sem_a = sl[10]
            g = (wid + t * _NW) * _CH
            pltpu.async_copy(src_hbm.at[pl.ds(g, _CH)], src_v, sem_a)
            pltpu.async_copy(dst_hbm.at[pl.ds(g, _CH)], dst_v, sem_a)
            pltpu.async_copy(et_hbm.at[pl.ds(g, _CH)], et_v, sem_a)
            pltpu.async_copy(
                ein_hbm.at[pl.ds((wid + t * _NW) * (_CH // 8), _CH // 8)],
                ein_v, sem_a)

        def wait_a(sl):
            src_v, dst_v, _, et_v, ein_v = sl[0], sl[1], sl[2], sl[3], sl[4]
            sem_a = sl[10]
            pltpu.make_async_copy(src_hbm.at[pl.ds(0, _CH)], src_v, sem_a).wait()
            pltpu.make_async_copy(dst_hbm.at[pl.ds(0, _CH)], dst_v, sem_a).wait()
            pltpu.make_async_copy(et_hbm.at[pl.ds(0, _CH)], et_v, sem_a).wait()
            pltpu.make_async_copy(
                ein_hbm.at[pl.ds(0, _CH // 8)], ein_v, sl[10]).wait()

        def issue_b(sl):
            src_v, dst_v, elg_v, erg_v, ftb = sl[0], sl[1], sl[5], sl[6], sl[8]
            sem_b = sl[11]
            pltpu.async_copy(el_hbm.at[src_v], elg_v, sem_b)
            pltpu.async_copy(er_hbm.at[dst_v], erg_v, sem_b)
            pltpu.async_copy(ft_hbm.at[src_v], ftb, sem_b)

        def wait_b(sl):
            src_v, dst_v, elg_v, erg_v, ftb = sl[0], sl[1], sl[5], sl[6], sl[8]
            sem_b = sl[11]
            pltpu.make_async_copy(el_hbm.at[src_v], elg_v, sem_b).wait()
            pltpu.make_async_copy(er_hbm.at[dst_v], erg_v, sem_b).wait()
            pltpu.make_async_copy(ft_hbm.at[src_v], ftb, sem_b).wait()

        iota = lax.iota(jnp.int32, _NL)
        one0 = jnp.where(iota == 0, jnp.float32(1), jnp.float32(0))

        def compute(sl):
            dst_v, dsts_v, et_v, ein_v = sl[1], sl[2], sl[3], sl[4]
            elg_v, erg_v, exb, ftb, st2 = sl[5], sl[6], sl[7], sl[8], sl[9]

            @plsc.parallel_loop(0, _CH, _NL, unroll=2)
            def _ex16(i):
                e = elg_v[pl.ds(i, _NL)] + erg_v[pl.ds(i, _NL)]
                e = e + et_v[pl.ds(i, _NL)]
                e = jnp.maximum(e, 0.2 * e)
                exb[pl.ds(i, _NL)] = jnp.exp(e)
                dsts_v[pl.ds(i, _NL)] = dst_v[pl.ds(i, _NL)]

            @plsc.parallel_loop(0, _CH, unroll=4)
            def _edge(i):
                bex = plsc.load_gather(exb, [jnp.broadcast_to(i, (_NL,))])
                for kk in range(_H // _NL):
                    slc = pl.ds(kk * _NL, _NL)
                    ftb[i, slc] = ftb[i, slc] * bex
                st2[i, pl.ds(0, _NL)] = (
                    ein_v[i // 8, pl.ds((i % 8) * _DE, _DE)] * bex)
                st2[i, pl.ds(_NL, _NL)] = bex * one0

        def issue_d(sl):
            dsts_v, ftb, st2, sem_d = sl[2], sl[8], sl[9], sl[12]
            pltpu.async_copy(ftb, acc128.at[dsts_v], sem_d, add=True)
            pltpu.async_copy(st2, acc32.at[dsts_v], sem_d, add=True)

        def wait_d(sl):
            dsts_v, ftb, st2, sem_d = sl[2], sl[8], sl[9], sl[12]
            pltpu.make_async_copy(ftb, acc128.at[dsts_v], sem_d).wait()
            pltpu.make_async_copy(st2, acc32.at[dsts_v], sem_d).wait()

        ftb0, st20 = slots[0][8], slots[0][9]

        @pl.loop(0, _CH)
        def _zrow(i):
            for kk in range(_H // _NL):
                ftb0[i, pl.ds(kk * _NL, _NL)] = zv
            st20[i, pl.ds(0, _NL)] = zv
            st20[i, pl.ds(_NL, _NL)] = zv

        r0 = s * _RPT
        for j in range(0, _RPT, _CH):
            nr = min(_CH, _RPT - j)
            pltpu.sync_copy(ftb0.at[pl.ds(0, nr)], acc128.at[pl.ds(r0 + j, nr)])
            pltpu.sync_copy(st20.at[pl.ds(0, nr)], acc32.at[pl.ds(r0 + j, nr)])
        plsc.subcore_barrier()

        issue_a(0, slots[0])
        issue_a(1, slots[1])
        wait_a(slots[0])
        issue_b(slots[0])

        @pl.loop(0, _TPT, step=2)
        def _pair(tt):
            for j in (0, 1):
                t = tt + j
                p, p1 = slots[j], slots[1 - j]
                if j == 0:
                    @pl.when(tt > 0)
                    def _():
                        wait_d(p1)
                else:
                    wait_d(p1)
                wait_a(p1)
                issue_b(p1)
                wait_b(p)
                compute(p)
                issue_d(p)
                issue_a(jnp.minimum(t + 2, _TPT - 1), p)

        wait_a(slots[1])
        wait_b(slots[0])
        wait_d(slots[1])

        plsc.subcore_barrier()
        pltpu.sync_copy(acc128.at[pl.ds(r0, _RPT)],
                        out128_hbm.at[c, pl.ds(r0, _RPT)])
        pltpu.sync_copy(acc32.at[pl.ds(r0, _RPT)],
                        out32_hbm.at[c, pl.ds(r0, _RPT)])

    return k(ft, el, er, eterm, ein2, src, dst)


def _layer(x, ein2, src, dst, W, We, al, ar, eterm, b, apply_tanh):
    ft, el, er = _dense_in(x, W, al, ar)
    acc128, acc32 = _edge_pass(ft, el, er, eterm, ein2, src, dst)
    return _combine(acc128, acc32, We, b, apply_tanh)


def kernel(node_in, edge_index, edge_in,
           W1, We1, al1, ar1, ae1, b1,
           W2, We2, al2, ar2, ae2, b2,
           W3, We3, al3, ar3, ae3, b3):
    npad = _EPAD - _E
    src = jnp.concatenate(
        [edge_index[0].astype(jnp.int32), jnp.zeros((npad,), jnp.int32)])
    dst = jnp.concatenate(
        [edge_index[1].astype(jnp.int32), jnp.zeros((npad,), jnp.int32)])
    ein2 = jnp.concatenate(
        [edge_in.reshape(_E // 8, 8 * _DE),
         jnp.zeros((npad // 8, 8 * _DE), jnp.float32)])
    etpad = jnp.full((npad,), -1e5, jnp.float32)
    et1 = jnp.concatenate([_eterm(edge_in, We1, ae1), etpad])
    et2 = jnp.concatenate([_eterm(edge_in, We2, ae2), etpad])
    et3 = jnp.concatenate([_eterm(edge_in, We3, ae3), etpad])
    h = _layer(node_in, ein2, src, dst, W1, We1, al1, ar1, et1, b1, True)
    h = _layer(h, ein2, src, dst, W2, We2, al2, ar2, et2, b2, True)
    h = _layer(h, ein2, src, dst, W2, We2, al2, ar2, et2, b2, True)
    h = _layer(h, ein2, src, dst, W3, We3, al3, ar3, et3, b3, False)
    return h


# speedup vs baseline: 1.1549x; 1.0013x over previous
import dataclasses
import functools

import jax
import jax.numpy as jnp
from jax import lax
from jax.experimental import pallas as pl
from jax.experimental.pallas import tpu as pltpu
from jax.experimental.pallas import tpu_sc as plsc

_N = 10000
_E = 320000
_D = 128
_DE = 16
_H = 128

_NC = 2
_NS = 16
_NL = 16
_NW = _NC * _NS
_CH = 64
_TPT = 158
_EPAD = _CH * _NW * _TPT
_NPAD = 10112
_RPT = _NPAD // _NS


def _dense_in(x, W, al, ar):
    def body(x_ref, w_ref, a_ref, ft_ref, el_ref, er_ref):
        ft = jnp.dot(x_ref[...], w_ref[...], preferred_element_type=jnp.float32)
        ft_ref[...] = ft
        el_ref[...] = jnp.dot(ft, a_ref[...][:, 0:1])
        er_ref[...] = jnp.dot(ft, a_ref[...][:, 1:2])

    a = jnp.stack([al, ar], axis=1)
    ft, el, er = pl.pallas_call(
        body,
        out_shape=(
            jax.ShapeDtypeStruct((_N, _H), jnp.float32),
            jax.ShapeDtypeStruct((_N, 1), jnp.float32),
            jax.ShapeDtypeStruct((_N, 1), jnp.float32),
        ),
    )(x, W, a)
    return ft, el[:, 0], er[:, 0]


def _eterm(edge_in, We, ae):
    def body(e_ref, w_ref, ae_ref, out_ref):
        w = jnp.dot(w_ref[...], ae_ref[...])
        wfull = jnp.concatenate([w] * 8, axis=0)
        ic = jax.lax.broadcasted_iota(jnp.int32, (8 * _DE, 8), 0)
        ik = jax.lax.broadcasted_iota(jnp.int32, (8 * _DE, 8), 1)
        wtile = jnp.where((ic // _DE) == ik, wfull, 0.0)
        out_ref[...] = jnp.dot(e_ref[...], wtile,
                               preferred_element_type=jnp.float32)

    out = pl.pallas_call(
        body,
        out_shape=jax.ShapeDtypeStruct((_E // 8, 8), jnp.float32),
    )(edge_in.reshape(_E // 8, 8 * _DE), We, ae.reshape(_H, 1))
    return out.reshape(_E)


def _combine(acc128, acc32, We, b, apply_tanh):
    def body(a128_ref, a32_ref, we_ref, b_ref, h_ref):
        num = a128_ref[0, :_N] + a128_ref[1, :_N]
        s32 = a32_ref[0, :_N] + a32_ref[1, :_N]
        se = s32[:, 0:_DE]
        den = s32[:, _DE:_DE + 1]
        h = (num + jnp.dot(se, we_ref[...], preferred_element_type=jnp.float32))
        h = h / jnp.maximum(den, 1e-9) + b_ref[...]
        if apply_tanh:
            h = jnp.tanh(h)
        h_ref[...] = h

    return pl.pallas_call(
        body,
        out_shape=jax.ShapeDtypeStruct((_N, _H), jnp.float32),
    )(acc128, acc32, We, b.reshape(1, _H))


def _edge_pass(ft, el, er, eterm, ein2, src, dst):
    mesh = plsc.VectorSubcoreMesh(core_axis_name="c", subcore_axis_name="s")
    cp = pltpu.CompilerParams(use_tc_tiling_on_sc=False)
    if "needs_layout_passes" in pltpu.CompilerParams.__dataclass_fields__:
        cp = dataclasses.replace(cp, needs_layout_passes=False)

    slot_scratch = [
        pltpu.VMEM((_CH,), jnp.int32),
        pltpu.VMEM((_CH,), jnp.int32),
        pltpu.VMEM((_CH,), jnp.int32),
        pltpu.VMEM((_CH,), jnp.float32),
        pltpu.VMEM((_CH // 8, _H), jnp.float32),
        pltpu.VMEM((_CH,), jnp.float32),
        pltpu.VMEM((_CH,), jnp.float32),
        pltpu.VMEM((_CH,), jnp.float32),
        pltpu.VMEM((_CH, _H), jnp.float32),
        pltpu.VMEM((_CH, 32), jnp.float32),
        pltpu.SemaphoreType.DMA,
        pltpu.SemaphoreType.DMA,
        pltpu.SemaphoreType.DMA,
    ]

    @functools.partial(
        pl.kernel,
        out_type=(jax.ShapeDtypeStruct((_NC, _NPAD, _H), jnp.float32),
                  jax.ShapeDtypeStruct((_NC, _NPAD, 32), jnp.float32)),
        mesh=mesh,
        compiler_params=cp,
        scratch_types=[
            pltpu.VMEM_SHARED((_NPAD, _H), jnp.float32),
            pltpu.VMEM_SHARED((_NPAD, 32), jnp.float32),
        ] + slot_scratch + slot_scratch,
    )
    def k(ft_hbm, el_hbm, er_hbm, et_hbm, ein_hbm, src_hbm, dst_hbm,
          out128_hbm, out32_hbm, acc128, acc32, *slotrefs):
        c = lax.axis_index("c")
        s = lax.axis_index("s")
        wid = s * _NC + c
        zv = jnp.zeros((_NL,), jnp.float32)
        nslot = len(slot_scratch)
        slots = [slotrefs[:nslot], slotrefs[nslot:]]

        def issue_a(t, sl):
            src_v, dst_v, _, et_v, ein_v = sl[0], sl[1], sl[2], sl[3], sl[4]
            sem_a = sl[10]
            g = (wid + t * _NW) * _CH
            pltpu.async_copy(src_hbm.at[pl.ds(g, _CH)], src_v, sem_a)
            pltpu.async_copy(dst_hbm.at[pl.ds(g, _CH)], dst_v, sem_a)
            pltpu.async_copy(et_hbm.at[pl.ds(g, _CH)], et_v, sem_a)
            pltpu.async_copy(
                ein_hbm.at[pl.ds((wid + t * _NW) * (_CH // 8), _CH // 8)],
                ein_v, sem_a)

        def wait_a(sl):
            src_v, dst_v, _, et_v, ein_v = sl[0], sl[1], sl[2], sl[3], sl[4]
            sem_a = sl[10]
            pltpu.make_async_copy(src_hbm.at[pl.ds(0, _CH)], src_v, sem_a).wait()
            pltpu.make_async_copy(dst_hbm.at[pl.ds(0, _CH)], dst_v, sem_a).wait()
            pltpu.make_async_copy(et_hbm.at[pl.ds(0, _CH)], et_v, sem_a).wait()
            pltpu.make_async_copy(
                ein_hbm.at[pl.ds(0, _CH // 8)], ein_v, sl[10]).wait()

        def issue_b(sl):
            src_v, dst_v, elg_v, erg_v, ftb = sl[0], sl[1], sl[5], sl[6], sl[8]
            sem_b = sl[11]
            pltpu.async_copy(el_hbm.at[src_v], elg_v, sem_b)
            pltpu.async_copy(er_hbm.at[dst_v], erg_v, sem_b)
            pltpu.async_copy(ft_hbm.at[src_v], ftb, sem_b)

        def wait_b(sl):
            src_v, dst_v, elg_v, erg_v, ftb = sl[0], sl[1], sl[5], sl[6], sl[8]
            sem_b = sl[11]
            pltpu.make_async_copy(el_hbm.at[src_v], elg_v, sem_b).wait()
            pltpu.make_async_copy(er_hbm.at[dst_v], erg_v, sem_b).wait()
            pltpu.make_async_copy(ft_hbm.at[src_v], ftb, sem_b).wait()

        iota = lax.iota(jnp.int32, _NL)
        one0 = jnp.where(iota == 0, jnp.float32(1), jnp.float32(0))

        def compute(sl):
            dst_v, dsts_v, et_v, ein_v = sl[1], sl[2], sl[3], sl[4]
            elg_v, erg_v, exb, ftb, st2 = sl[5], sl[6], sl[7], sl[8], sl[9]

            @plsc.parallel_loop(0, _CH, _NL, unroll=4)
            def _ex16(i):
                e = elg_v[pl.ds(i, _NL)] + erg_v[pl.ds(i, _NL)]
                e = e + et_v[pl.ds(i, _NL)]
                e = jnp.maximum(e, 0.2 * e)
                exb[pl.ds(i, _NL)] = jnp.exp(e)
                dsts_v[pl.ds(i, _NL)] = dst_v[pl.ds(i, _NL)]

            @plsc.parallel_loop(0, _CH, unroll=8)
            def _edge(i):
                bex = plsc.load_gather(exb, [jnp.broadcast_to(i, (_NL,))])
                for kk in range(_H // _NL):
                    slc = pl.ds(kk * _NL, _NL)
                    ftb[i, slc] = ftb[i, slc] * bex
                st2[i, pl.ds(0, _NL)] = (
                    ein_v[i // 8, pl.ds((i % 8) * _DE, _DE)] * bex)
                st2[i, pl.ds(_NL, _NL)] = bex * one0

        def issue_d(sl):
            dsts_v, ftb, st2, sem_d = sl[2], sl[8], sl[9], sl[12]
            pltpu.async_copy(ftb, acc128.at[dsts_v], sem_d, add=True)
            pltpu.async_copy(st2, acc32.at[dsts_v], sem_d, add=True)

        def wait_d(sl):
            dsts_v, ftb, st2, sem_d = sl[2], sl[8], sl[9], sl[12]
            pltpu.make_async_copy(ftb, acc128.at[dsts_v], sem_d).wait()
            pltpu.make_async_copy(st2, acc32.at[dsts_v], sem_d).wait()

        ftb0, st20 = slots[0][8], slots[0][9]

        @pl.loop(0, _CH)
        def _zrow(i):
            for kk in range(_H // _NL):
                ftb0[i, pl.ds(kk * _NL, _NL)] = zv
            st20[i, pl.ds(0, _NL)] = zv
            st20[i, pl.ds(_NL, _NL)] = zv

        r0 = s * _RPT
        for j in range(0, _RPT, _CH):
            nr = min(_CH, _RPT - j)
            pltpu.sync_copy(ftb0.at[pl.ds(0, nr)], acc128.at[pl.ds(r0 + j, nr)])
            pltpu.sync_copy(st20.at[pl.ds(0, nr)], acc32.at[pl.ds(r0 + j, nr)])
        plsc.subcore_barrier()

        issue_a(0, slots[0])
        issue_a(1, slots[1])
        wait_a(slots[0])
        issue_b(slots[0])

        @pl.loop(0, _TPT, step=2)
        def _pair(tt):
            for j in (0, 1):
                t = tt + j
                p, p1 = slots[j], slots[1 - j]
                if j == 0:
                    @pl.when(tt > 0)
                    def _():
                        wait_d(p1)
                else:
                    wait_d(p1)
                wait_a(p1)
                issue_b(p1)
                wait_b(p)
                compute(p)
                issue_d(p)
                issue_a(jnp.minimum(t + 2, _TPT - 1), p)

        wait_a(slots[1])
        wait_b(slots[0])
        wait_d(slots[1])

        plsc.subcore_barrier()
        pltpu.sync_copy(acc128.at[pl.ds(r0, _RPT)],
                        out128_hbm.at[c, pl.ds(r0, _RPT)])
        pltpu.sync_copy(acc32.at[pl.ds(r0, _RPT)],
                        out32_hbm.at[c, pl.ds(r0, _RPT)])

    return k(ft, el, er, eterm, ein2, src, dst)


def _layer(x, ein2, src, dst, W, We, al, ar, eterm, b, apply_tanh):
    ft, el, er = _dense_in(x, W, al, ar)
    acc128, acc32 = _edge_pass(ft, el, er, eterm, ein2, src, dst)
    return _combine(acc128, acc32, We, b, apply_tanh)


def kernel(node_in, edge_index, edge_in,
           W1, We1, al1, ar1, ae1, b1,
           W2, We2, al2, ar2, ae2, b2,
           W3, We3, al3, ar3, ae3, b3):
    npad = _EPAD - _E
    src = jnp.concatenate(
        [edge_index[0].astype(jnp.int32), jnp.zeros((npad,), jnp.int32)])
    dst = jnp.concatenate(
        [edge_index[1].astype(jnp.int32), jnp.zeros((npad,), jnp.int32)])
    ein2 = jnp.concatenate(
        [edge_in.reshape(_E // 8, 8 * _DE),
         jnp.zeros((npad // 8, 8 * _DE), jnp.float32)])
    etpad = jnp.full((npad,), -1e5, jnp.float32)
    et1 = jnp.concatenate([_eterm(edge_in, We1, ae1), etpad])
    et2 = jnp.concatenate([_eterm(edge_in, We2, ae2), etpad])
    et3 = jnp.concatenate([_eterm(edge_in, We3, ae3), etpad])
    h = _layer(node_in, ein2, src, dst, W1, We1, al1, ar1, et1, b1, True)
    h = _layer(h, ein2, src, dst, W2, We2, al2, ar2, et2, b2, True)
    h = _layer(h, ein2, src, dst, W2, We2, al2, ar2, et2, b2, True)
    h = _layer(h, ein2, src, dst, W3, We3, al3, ar3, et3, b3, False)
    return h


# speedup vs baseline: 1.3510x; 1.1698x over previous
import dataclasses
import functools

import jax
import jax.numpy as jnp
from jax import lax
from jax.experimental import pallas as pl
from jax.experimental.pallas import tpu as pltpu
from jax.experimental.pallas import tpu_sc as plsc

_N = 10000
_E = 320000
_D = 128
_DE = 16
_H = 128

_NC = 2
_NS = 16
_NL = 16
_NW = _NC * _NS
_CH = 64
_TPT = 158
_EPAD = _CH * _NW * _TPT
_NPAD = 10112
_RPT = _NPAD // _NS


def _dense_in(x, W, al, ar):
    def body(x_ref, w_ref, a_ref, ftbf_ref, el_ref, er_ref):
        ft = jnp.dot(x_ref[...], w_ref[...], preferred_element_type=jnp.float32)
        words = []
        for kk in range(_H // 32):
            a32 = ft[:, kk * 32:kk * 32 + 16]
            b32 = ft[:, kk * 32 + 16:kk * 32 + 32]
            ai = lax.bitcast_convert_type(a32, jnp.int32)
            bi = lax.bitcast_convert_type(b32, jnp.int32)
            ar_ = ai + 0x7FFF + (lax.shift_right_logical(ai, 16) & 1)
            br_ = bi + 0x7FFF + (lax.shift_right_logical(bi, 16) & 1)
            lo = lax.shift_right_logical(ar_, 16)
            hi = lax.shift_left(lax.shift_right_logical(br_, 16), 16)
            words.append(lo | hi)
        ftbf_ref[...] = jnp.concatenate(words, axis=1)
        el_ref[...] = jnp.dot(ft, a_ref[...][:, 0:1])
        er_ref[...] = jnp.dot(ft, a_ref[...][:, 1:2])

    a = jnp.stack([al, ar], axis=1)
    ftbf, el, er = pl.pallas_call(
        body,
        out_shape=(
            jax.ShapeDtypeStruct((_N, _H // 2), jnp.int32),
            jax.ShapeDtypeStruct((_N, 1), jnp.float32),
            jax.ShapeDtypeStruct((_N, 1), jnp.float32),
        ),
    )(x, W, a)
    return ftbf, el[:, 0], er[:, 0]


def _eterm(edge_in, We, ae):
    def body(e_ref, w_ref, ae_ref, out_ref):
        w = jnp.dot(w_ref[...], ae_ref[...])
        wfull = jnp.concatenate([w] * 8, axis=0)
        ic = jax.lax.broadcasted_iota(jnp.int32, (8 * _DE, 8), 0)
        ik = jax.lax.broadcasted_iota(jnp.int32, (8 * _DE, 8), 1)
        wtile = jnp.where((ic // _DE) == ik, wfull, 0.0)
        out_ref[...] = jnp.dot(e_ref[...], wtile,
                               preferred_element_type=jnp.float32)

    out = pl.pallas_call(
        body,
        out_shape=jax.ShapeDtypeStruct((_E // 8, 8), jnp.float32),
    )(edge_in.reshape(_E // 8, 8 * _DE), We, ae.reshape(_H, 1))
    return out.reshape(_E)


def _combine(acc128, acc32, We, b, apply_tanh):
    def body(a128_ref, a32_ref, we_ref, b_ref, h_ref):
        num = a128_ref[0, :_N] + a128_ref[1, :_N]
        s32 = a32_ref[0, :_N] + a32_ref[1, :_N]
        se = s32[:, 0:_DE]
        den = s32[:, _DE:_DE + 1]
        h = (num + jnp.dot(se, we_ref[...], preferred_element_type=jnp.float32))
        h = h / jnp.maximum(den, 1e-9) + b_ref[...]
        if apply_tanh:
            h = jnp.tanh(h)
        h_ref[...] = h

    return pl.pallas_call(
        body,
        out_shape=jax.ShapeDtypeStruct((_N, _H), jnp.float32),
    )(acc128, acc32, We, b.reshape(1, _H))


def _edge_pass(ft, el, er, eterm, ein2, src, dst):
    mesh = plsc.VectorSubcoreMesh(core_axis_name="c", subcore_axis_name="s")
    cp = pltpu.CompilerParams(use_tc_tiling_on_sc=False)
    if "needs_layout_passes" in pltpu.CompilerParams.__dataclass_fields__:
        cp = dataclasses.replace(cp, needs_layout_passes=False)

    slot_scratch = [
        pltpu.VMEM((_CH,), jnp.int32),
        pltpu.VMEM((_CH,), jnp.int32),
        pltpu.VMEM((_CH,), jnp.int32),
        pltpu.VMEM((_CH,), jnp.float32),
        pltpu.VMEM((_CH // 8, _H), jnp.float32),
        pltpu.VMEM((_CH,), jnp.float32),
        pltpu.VMEM((_CH,), jnp.float32),
        pltpu.VMEM((_CH,), jnp.float32),
        pltpu.VMEM((_CH, _H // 2), jnp.int32),
        pltpu.VMEM((_CH, 32), jnp.float32),
        pltpu.SemaphoreType.DMA,
        pltpu.SemaphoreType.DMA,
        pltpu.SemaphoreType.DMA,
    ]

    @functools.partial(
        pl.kernel,
        out_type=(jax.ShapeDtypeStruct((_NC, _NPAD, _H), jnp.float32),
                  jax.ShapeDtypeStruct((_NC, _NPAD, 32), jnp.float32)),
        mesh=mesh,
        compiler_params=cp,
        scratch_types=[
            pltpu.VMEM_SHARED((_NPAD, _H), jnp.float32),
            pltpu.VMEM_SHARED((_NPAD, 32), jnp.float32),
        ] + slot_scratch + slot_scratch + [
            pltpu.VMEM((_CH, _H), jnp.float32),
        ],
    )
    def k(ft_hbm, el_hbm, er_hbm, et_hbm, ein_hbm, src_hbm, dst_hbm,
          out128_hbm, out32_hbm, acc128, acc32, *slotrefs):
        c = lax.axis_index("c")
        s = lax.axis_index("s")
        wid = s * _NC + c
        zv = jnp.zeros((_NL,), jnp.float32)
        nslot = len(slot_scratch)
        slots = [slotrefs[:nslot], slotrefs[nslot:2 * nslot]]
        stg = slotrefs[2 * nslot]

        def issue_a(t, sl):
            src_v, dst_v, _, et_v, ein_v = sl[0], sl[1], sl[2], sl[3], sl[4]
            sem_a = sl[10]
            g = (wid + t * _NW) * _CH
            pltpu.async_copy(src_hbm.at[pl.ds(g, _CH)], src_v, sem_a)
            pltpu.async_copy(dst_hbm.at[pl.ds(g, _CH)], dst_v, sem_a)
            pltpu.async_copy(et_hbm.at[pl.ds(g, _CH)], et_v, sem_a)
            pltpu.async_copy(
                ein_hbm.at[pl.ds((wid + t * _NW) * (_CH // 8), _CH // 8)],
                ein_v, sem_a)

        def wait_a(sl):
            src_v, dst_v, _, et_v, ein_v = sl[0], sl[1], sl[2], sl[3], sl[4]
            sem_a = sl[10]
            pltpu.make_async_copy(src_hbm.at[pl.ds(0, _CH)], src_v, sem_a).wait()
            pltpu.make_async_copy(dst_hbm.at[pl.ds(0, _CH)], dst_v, sem_a).wait()
            pltpu.make_async_copy(et_hbm.at[pl.ds(0, _CH)], et_v, sem_a).wait()
            pltpu.make_async_copy(
                ein_hbm.at[pl.ds(0, _CH // 8)], ein_v, sl[10]).wait()

        def issue_b(sl):
            src_v, dst_v, elg_v, erg_v, ftb = sl[0], sl[1], sl[5], sl[6], sl[8]
            sem_b = sl[11]
            pltpu.async_copy(el_hbm.at[src_v], elg_v, sem_b)
            pltpu.async_copy(er_hbm.at[dst_v], erg_v, sem_b)
            pltpu.async_copy(ft_hbm.at[src_v], ftb, sem_b)

        def wait_b(sl):
            src_v, dst_v, elg_v, erg_v, ftb = sl[0], sl[1], sl[5], sl[6], sl[8]
            sem_b = sl[11]
            pltpu.make_async_copy(el_hbm.at[src_v], elg_v, sem_b).wait()
            pltpu.make_async_copy(er_hbm.at[dst_v], erg_v, sem_b).wait()
            pltpu.make_async_copy(ft_hbm.at[src_v], ftb, sem_b).wait()

        iota = lax.iota(jnp.int32, _NL)
        one0 = jnp.where(iota == 0, jnp.float32(1), jnp.float32(0))

        def compute(sl):
            dst_v, dsts_v, et_v, ein_v = sl[1], sl[2], sl[3], sl[4]
            elg_v, erg_v, exb, ftb, st2 = sl[5], sl[6], sl[7], sl[8], sl[9]

            @plsc.parallel_loop(0, _CH, _NL, unroll=4)
            def _ex16(i):
                e = elg_v[pl.ds(i, _NL)] + erg_v[pl.ds(i, _NL)]
                e = e + et_v[pl.ds(i, _NL)]
                e = jnp.maximum(e, 0.2 * e)
                exb[pl.ds(i, _NL)] = jnp.exp(e)
                dsts_v[pl.ds(i, _NL)] = dst_v[pl.ds(i, _NL)]

            @plsc.parallel_loop(0, _CH, unroll=8)
            def _edge(i):
                bex = plsc.load_gather(exb, [jnp.broadcast_to(i, (_NL,))])
                for kk in range(_H // 32):
                    w = ftb[i, pl.ds(kk * _NL, _NL)]
                    lo = plsc.bitcast(lax.shift_left(w, 16), jnp.float32)
                    hi = plsc.bitcast(w & jnp.int32(-65536), jnp.float32)
                    stg[i, pl.ds(kk * 32, _NL)] = lo * bex
                    stg[i, pl.ds(kk * 32 + _NL, _NL)] = hi * bex
                st2[i, pl.ds(0, _NL)] = (
                    ein_v[i // 8, pl.ds((i % 8) * _DE, _DE)] * bex)
                st2[i, pl.ds(_NL, _NL)] = bex * one0

        def issue_d(sl):
            dsts_v, st2, sem_d = sl[2], sl[9], sl[12]
            pltpu.sync_copy(stg, acc128.at[dsts_v], add=True)
            pltpu.async_copy(st2, acc32.at[dsts_v], sem_d, add=True)

        def wait_d(sl):
            dsts_v, st2, sem_d = sl[2], sl[9], sl[12]
            pltpu.make_async_copy(st2, acc32.at[dsts_v], sem_d).wait()

        st20 = slots[0][9]

        @pl.loop(0, _CH)
        def _zrow(i):
            for kk in range(_H // _NL):
                stg[i, pl.ds(kk * _NL, _NL)] = zv
            st20[i, pl.ds(0, _NL)] = zv
            st20[i, pl.ds(_NL, _NL)] = zv

        r0 = s * _RPT
        for j in range(0, _RPT, _CH):
            nr = min(_CH, _RPT - j)
            pltpu.sync_copy(stg.at[pl.ds(0, nr)], acc128.at[pl.ds(r0 + j, nr)])
            pltpu.sync_copy(st20.at[pl.ds(0, nr)], acc32.at[pl.ds(r0 + j, nr)])
        plsc.subcore_barrier()

        issue_a(0, slots[0])
        issue_a(1, slots[1])
        wait_a(slots[0])
        issue_b(slots[0])

        @pl.loop(0, _TPT, step=2)
        def _pair(tt):
            for j in (0, 1):
                t = tt + j
                p, p1 = slots[j], slots[1 - j]
                if j == 0:
                    @pl.when(tt > 0)
                    def _():
                        wait_d(p1)
                else:
                    wait_d(p1)
                wait_a(p1)
                issue_b(p1)
                wait_b(p)
                compute(p)
                issue_d(p)
                issue_a(jnp.minimum(t + 2, _TPT - 1), p)

        wait_a(slots[1])
        wait_b(slots[0])
        wait_d(slots[1])

        plsc.subcore_barrier()
        pltpu.sync_copy(acc128.at[pl.ds(r0, _RPT)],
                        out128_hbm.at[c, pl.ds(r0, _RPT)])
        pltpu.sync_copy(acc32.at[pl.ds(r0, _RPT)],
                        out32_hbm.at[c, pl.ds(r0, _RPT)])

    return k(ft, el, er, eterm, ein2, src, dst)


def _layer(x, ein2, src, dst, W, We, al, ar, eterm, b, apply_tanh):
    ft, el, er = _dense_in(x, W, al, ar)
    acc128, acc32 = _edge_pass(ft, el, er, eterm, ein2, src, dst)
    return _combine(acc128, acc32, We, b, apply_tanh)


def kernel(node_in, edge_index, edge_in,
           W1, We1, al1, ar1, ae1, b1,
           W2, We2, al2, ar2, ae2, b2,
           W3, We3, al3, ar3, ae3, b3):
    npad = _EPAD - _E
    src = jnp.concatenate(
        [edge_index[0].astype(jnp.int32), jnp.zeros((npad,), jnp.int32)])
    dst = jnp.concatenate(
        [edge_index[1].astype(jnp.int32), jnp.zeros((npad,), jnp.int32)])
    ein2 = jnp.concatenate(
        [edge_in.reshape(_E // 8, 8 * _DE),
         jnp.zeros((npad // 8, 8 * _DE), jnp.float32)])
    etpad = jnp.full((npad,), -1e5, jnp.float32)
    et1 = jnp.concatenate([_eterm(edge_in, We1, ae1), etpad])
    et2 = jnp.concatenate([_eterm(edge_in, We2, ae2), etpad])
    et3 = jnp.concatenate([_eterm(edge_in, We3, ae3), etpad])
    h = _layer(node_in, ein2, src, dst, W1, We1, al1, ar1, et1, b1, True)
    h = _layer(h, ein2, src, dst, W2, We2, al2, ar2, et2, b2, True)
    h = _layer(h, ein2, src, dst, W2, We2, al2, ar2, et2, b2, True)
    h = _layer(h, ein2, src, dst, W3, We3, al3, ar3, et3, b3, False)
    return h
